# baseline (speedup 1.0000x reference)
import jax
import jax.numpy as jnp
from jax.experimental import pallas as pl
from jax.experimental.pallas import tpu as pltpu

P = 8
FB = 2048


def _ff_kernel(x_ref, w1a_ref, w1b_ref, b1_ref, w2a_ref, w2b_ref, b2_ref,
               o_ref):
    f = pl.program_id(2)
    x = x_ref[0]
    hd = w1a_ref.shape[1]
    hf = w2a_ref.shape[1]
    h = jnp.dot(x[:, :hd], w1a_ref[0], preferred_element_type=jnp.float32)
    h += jnp.dot(x[:, hd:], w1b_ref[0], preferred_element_type=jnp.float32)
    h = jax.nn.gelu(h + b1_ref[0])
    y = jnp.dot(h[:, :hf], w2a_ref[0], preferred_element_type=jnp.float32)
    y += jnp.dot(h[:, hf:], w2b_ref[0], preferred_element_type=jnp.float32)

    @pl.when(f == 0)
    def _init():
        o_ref[0] = y + b2_ref[0]

    @pl.when(f != 0)
    def _acc():
        o_ref[0] += y


def kernel(x, phases, W1, b1, W2, b2):
    del phases
    B, S, D = x.shape
    _, _, F = W1.shape
    TB = S // P
    nf = F // FB
    HD = D // 2
    HF = FB // 2
    b1r = b1.reshape(P, 1, F)
    b2r = b2.reshape(P, 1, D)

    grid = (B, P, nf)
    out = pl.pallas_call(
        _ff_kernel,
        grid=grid,
        in_specs=[
            pl.BlockSpec((1, TB, D), lambda b, p, f: (b, p, 0)),
            pl.BlockSpec((1, HD, FB), lambda b, p, f: (p, 0, f)),
            pl.BlockSpec((1, HD, FB), lambda b, p, f: (p, 1, f)),
            pl.BlockSpec((1, 1, FB), lambda b, p, f: (p, 0, f)),
            pl.BlockSpec((1, HF, D), lambda b, p, f: (p, 2 * f, 0)),
            pl.BlockSpec((1, HF, D), lambda b, p, f: (p, 2 * f + 1, 0)),
            pl.BlockSpec((1, 1, D), lambda b, p, f: (p, 0, 0)),
        ],
        out_specs=pl.BlockSpec((1, TB, D), lambda b, p, f: (b, p, 0)),
        out_shape=jax.ShapeDtypeStruct((B, S, D), x.dtype),
        compiler_params=pltpu.CompilerParams(
            dimension_semantics=("parallel", "parallel", "arbitrary")),
    )(x, W1, W1, b1r, W2, W2, b2r)
    return out


# speedup vs baseline: 1.0010x; 1.0010x over previous
import jax
import jax.numpy as jnp
from jax.experimental import pallas as pl
from jax.experimental.pallas import tpu as pltpu

P = 8
FB = 2048


def _ff_kernel(x_ref, w1a_ref, w1b_ref, b1_ref, w2a_ref, w2b_ref, b2_ref,
               o_ref):
    f = pl.program_id(2)
    x = x_ref[0]
    hf = FB // 2
    ha = jnp.dot(x, w1a_ref[0], preferred_element_type=jnp.float32)
    hb = jnp.dot(x, w1b_ref[0], preferred_element_type=jnp.float32)
    ha = jax.nn.gelu(ha + b1_ref[0, :, :hf])
    hb = jax.nn.gelu(hb + b1_ref[0, :, hf:])
    y = jnp.dot(ha, w2a_ref[0], preferred_element_type=jnp.float32)
    y += jnp.dot(hb, w2b_ref[0], preferred_element_type=jnp.float32)

    @pl.when(f == 0)
    def _init():
        o_ref[0] = y + b2_ref[0]

    @pl.when(f != 0)
    def _acc():
        o_ref[0] += y


def kernel(x, phases, W1, b1, W2, b2):
    del phases
    B, S, D = x.shape
    _, _, F = W1.shape
    TB = S // P
    nf = F // FB
    HF = FB // 2
    b1r = b1.reshape(P, 1, F)
    b2r = b2.reshape(P, 1, D)

    grid = (B, P, nf)
    out = pl.pallas_call(
        _ff_kernel,
        grid=grid,
        in_specs=[
            pl.BlockSpec((1, TB, D), lambda b, p, f: (b, p, 0)),
            pl.BlockSpec((1, D, HF), lambda b, p, f: (p, 0, 2 * f)),
            pl.BlockSpec((1, D, HF), lambda b, p, f: (p, 0, 2 * f + 1)),
            pl.BlockSpec((1, 1, FB), lambda b, p, f: (p, 0, f)),
            pl.BlockSpec((1, HF, D), lambda b, p, f: (p, 2 * f, 0)),
            pl.BlockSpec((1, HF, D), lambda b, p, f: (p, 2 * f + 1, 0)),
            pl.BlockSpec((1, 1, D), lambda b, p, f: (p, 0, 0)),
        ],
        out_specs=pl.BlockSpec((1, TB, D), lambda b, p, f: (b, p, 0)),
        out_shape=jax.ShapeDtypeStruct((B, S, D), x.dtype),
        compiler_params=pltpu.CompilerParams(
            dimension_semantics=("parallel", "parallel", "arbitrary")),
    )(x, W1, W1, b1r, W2, W2, b2r)
    return out
